# baseline (speedup 1.0000x reference)
import functools

import jax
import jax.numpy as jnp
from jax import lax
from jax.experimental import pallas as pl
from jax.experimental.pallas import tpu as pltpu
from jax.experimental.pallas import tpu_sc as plsc

B = 16384
D = 64
HID = 64
NE = 100000
NC, NS = 2, 16
NW = NC * NS
BPW = B // NW
CH = 128
NCH = BPW // CH
L = 16

_mesh = plsc.VectorSubcoreMesh(core_axis_name="c", subcore_axis_name="s")


def _worker_base():
    wid = lax.axis_index("s") * NC + lax.axis_index("c")
    return wid * BPW


@functools.partial(
    pl.kernel,
    out_type=jax.ShapeDtypeStruct((B,), jnp.float32),
    mesh=_mesh,
    scratch_types=[
        pltpu.VMEM((BPW,), jnp.int32),
        pltpu.VMEM((BPW,), jnp.int32),
        pltpu.VMEM((BPW,), jnp.int32),
        pltpu.VMEM((BPW,), jnp.int32),
        pltpu.VMEM((2, CH, 2 * D), jnp.float32),
        pltpu.VMEM((2, CH, 2 * D), jnp.float32),
        pltpu.VMEM((HID,), jnp.float32),
        pltpu.VMEM((L,), jnp.float32),
        pltpu.VMEM((BPW,), jnp.float32),
        pltpu.SemaphoreType.DMA,
        pltpu.SemaphoreType.DMA,
    ],
    compiler_params=pltpu.CompilerParams(use_tc_tiling_on_sc=False,
                                         needs_layout_passes=False),
)
def _sc_score(sim_hbm, edges_hbm, p_hbm, w2_hbm, b2_hbm, out_hbm,
              eidx_m, eidx_d, cm_v, cd_v, rows_m, rows_d, w2_v, b2_v, out_v,
              sem_m, sem_d):
    base = _worker_base()
    pltpu.sync_copy(edges_hbm.at[0, pl.ds(base, BPW)], eidx_m)
    pltpu.sync_copy(edges_hbm.at[1, pl.ds(base, BPW)], eidx_d)
    pltpu.sync_copy(w2_hbm, w2_v)
    pltpu.sync_copy(b2_hbm, b2_v)
    pend = []
    for j in range(NCH):
        s = pl.ds(j * CH, CH)
        pend.append(pltpu.async_copy(sim_hbm.at[eidx_m.at[s]], cm_v.at[s], sem_m))
        pend.append(pltpu.async_copy(sim_hbm.at[eidx_d.at[s]], cd_v.at[s], sem_d))
    for h in pend:
        h.wait()

    def fire(j):
        s = pl.ds(j * CH, CH)
        return (pltpu.async_copy(p_hbm.at[cm_v.at[s]], rows_m.at[j % 2], sem_m),
                pltpu.async_copy(p_hbm.at[cd_v.at[s]], rows_d.at[j % 2], sem_d))

    lane = lax.iota(jnp.int32, L)
    pend = fire(0)
    for j in range(NCH):
        for h in pend:
            h.wait()
        if j + 1 < NCH:
            nxt = fire(j + 1)
        buf = j % 2

        def group(g, carry):
            z = jnp.zeros((L,), jnp.float32)
            for p in range(L):
                e = g * L + p
                acc = jnp.zeros((L,), jnp.float32)
                for k in range(D // L):
                    hm = rows_m[buf, e, pl.ds(k * L, L)]
                    hd = rows_d[buf, e, pl.ds(D + k * L, L)]
                    hv = jnp.maximum(hm + hd, 0.0)
                    acc = acc + hv * w2_v[pl.ds(k * L, L)]
                z = jnp.where(lane == p, jnp.sum(acc), z)
            zv = z + b2_v[...]
            out_v[pl.ds(j * CH + g * L, L)] = 1.0 / (1.0 + jnp.exp(-zv))
            return carry

        lax.fori_loop(0, CH // L, group, 0)
        if j + 1 < NCH:
            pend = nxt
    pltpu.sync_copy(out_v, out_hbm.at[pl.ds(base, BPW)])


BLKR = 10000


def _proj_body(em_hbm, ed_hbm, w1bd_ref, b1p_ref, p_ref, em_v, ed_v, sem):
    i = pl.program_id(0)
    cm = pltpu.make_async_copy(em_hbm.at[pl.ds(i * BLKR, BLKR)], em_v, sem)
    cm.start()
    cd = pltpu.make_async_copy(ed_hbm.at[pl.ds(i * BLKR, BLKR)], ed_v, sem)
    cd.start()
    cm.wait()
    cd.wait()
    x = jnp.concatenate([em_v[...], ed_v[...]], axis=1).astype(jnp.bfloat16)
    p_ref[...] = (jnp.dot(x, w1bd_ref[...], preferred_element_type=jnp.float32)
                  + b1p_ref[...])


_proj = pl.pallas_call(
    _proj_body,
    grid=(NE // BLKR,),
    in_specs=[
        pl.BlockSpec(memory_space=pl.ANY),
        pl.BlockSpec(memory_space=pl.ANY),
        pl.BlockSpec((2 * D, 2 * D), lambda i: (0, 0)),
        pl.BlockSpec((1, 2 * D), lambda i: (0, 0)),
    ],
    out_specs=pl.BlockSpec((BLKR, 2 * D), lambda i: (i, 0)),
    out_shape=jax.ShapeDtypeStruct((NE, 2 * D), jnp.float32),
    scratch_shapes=[
        pltpu.VMEM((BLKR, D), jnp.float32),
        pltpu.VMEM((BLKR, D), jnp.float32),
        pltpu.SemaphoreType.DMA,
    ],
)


def kernel(sim_data, train_data, Em_table, Ed_table, W1, b1, W2, b2):
    edges = train_data.T
    w1bd = jnp.zeros((2 * D, 2 * D), jnp.bfloat16)
    w1bd = w1bd.at[:D, :HID].set(W1[:D].astype(jnp.bfloat16))
    w1bd = w1bd.at[D:, HID:].set(W1[D:].astype(jnp.bfloat16))
    b1p = jnp.concatenate([b1, jnp.zeros((HID,), jnp.float32)]).reshape(1, 2 * D)
    p = _proj(Em_table, Ed_table, w1bd, b1p)
    return _sc_score(sim_data, edges, p, W2.reshape(HID),
                     jnp.broadcast_to(b2, (L,)))


# speedup vs baseline: 1.1700x; 1.1700x over previous
import functools

import jax
import jax.numpy as jnp
from jax import lax
from jax.experimental import pallas as pl
from jax.experimental.pallas import tpu as pltpu
from jax.experimental.pallas import tpu_sc as plsc

B = 16384
D = 64
HID = 64
NE = 100000
NC, NS = 2, 16
NW = NC * NS
BPW = B // NW
CH = 128
NCH = BPW // CH
L = 16

_mesh = plsc.VectorSubcoreMesh(core_axis_name="c", subcore_axis_name="s")


def _worker_base():
    wid = lax.axis_index("s") * NC + lax.axis_index("c")
    return wid * BPW


@functools.partial(
    pl.kernel,
    out_type=jax.ShapeDtypeStruct((B,), jnp.float32),
    mesh=_mesh,
    scratch_types=[
        pltpu.VMEM((BPW,), jnp.int32),
        pltpu.VMEM((BPW,), jnp.int32),
        pltpu.VMEM((BPW,), jnp.int32),
        pltpu.VMEM((BPW,), jnp.int32),
        pltpu.VMEM((2, CH, 2 * D), jnp.float32),
        pltpu.VMEM((2, CH, 2 * D), jnp.float32),
        pltpu.VMEM((HID,), jnp.float32),
        pltpu.VMEM((L,), jnp.float32),
        pltpu.VMEM((BPW,), jnp.float32),
        pltpu.SemaphoreType.DMA,
        pltpu.SemaphoreType.DMA,
    ],
    compiler_params=pltpu.CompilerParams(use_tc_tiling_on_sc=False,
                                         needs_layout_passes=False),
)
def _sc_score(sim_hbm, edges_hbm, p_hbm, w2_hbm, b2_hbm, out_hbm,
              eidx_m, eidx_d, cm_v, cd_v, rows_m, rows_d, w2_v, b2_v, out_v,
              sem_m, sem_d):
    base = _worker_base()
    pltpu.sync_copy(edges_hbm.at[0, pl.ds(base, BPW)], eidx_m)
    pltpu.sync_copy(edges_hbm.at[1, pl.ds(base, BPW)], eidx_d)
    pltpu.sync_copy(w2_hbm, w2_v)
    pltpu.sync_copy(b2_hbm, b2_v)
    pend = []
    for j in range(NCH):
        s = pl.ds(j * CH, CH)
        pend.append(pltpu.async_copy(sim_hbm.at[eidx_m.at[s]], cm_v.at[s], sem_m))
        pend.append(pltpu.async_copy(sim_hbm.at[eidx_d.at[s]], cd_v.at[s], sem_d))
    for h in pend:
        h.wait()

    def fire(j):
        s = pl.ds(j * CH, CH)
        return (pltpu.async_copy(p_hbm.at[cm_v.at[s]], rows_m.at[j % 2], sem_m),
                pltpu.async_copy(p_hbm.at[cd_v.at[s]], rows_d.at[j % 2], sem_d))

    lane = lax.iota(jnp.int32, L)
    pend = fire(0)
    for j in range(NCH):
        for h in pend:
            h.wait()
        if j + 1 < NCH:
            nxt = fire(j + 1)
        buf = j % 2

        def group(g, carry):
            z = jnp.zeros((L,), jnp.float32)
            for p in range(L):
                e = g * L + p
                acc = jnp.zeros((L,), jnp.float32)
                for k in range(D // L):
                    hm = rows_m[buf, e, pl.ds(k * L, L)]
                    hd = rows_d[buf, e, pl.ds(D + k * L, L)]
                    hv = jnp.maximum(hm + hd, 0.0)
                    acc = acc + hv * w2_v[pl.ds(k * L, L)]
                z = jnp.where(lane == p, jnp.sum(acc), z)
            zv = z + b2_v[...]
            out_v[pl.ds(j * CH + g * L, L)] = 1.0 / (1.0 + jnp.exp(-zv))
            return carry

        lax.fori_loop(0, CH // L, group, 0)
        if j + 1 < NCH:
            pend = nxt
    pltpu.sync_copy(out_v, out_hbm.at[pl.ds(base, BPW)])


BLKR = 10000


def _proj_body(em_ref, ed_ref, w1bd_ref, b1p_ref, p_ref):
    x = jnp.concatenate([em_ref[...], ed_ref[...]], axis=1).astype(jnp.bfloat16)
    p_ref[...] = (jnp.dot(x, w1bd_ref[...], preferred_element_type=jnp.float32)
                  + b1p_ref[...])


_proj = pl.pallas_call(
    _proj_body,
    grid=(NE // BLKR,),
    in_specs=[
        pl.BlockSpec((BLKR, D), lambda i: (i, 0)),
        pl.BlockSpec((BLKR, D), lambda i: (i, 0)),
        pl.BlockSpec((2 * D, 2 * D), lambda i: (0, 0)),
        pl.BlockSpec((1, 2 * D), lambda i: (0, 0)),
    ],
    out_specs=pl.BlockSpec((BLKR, 2 * D), lambda i: (i, 0)),
    out_shape=jax.ShapeDtypeStruct((NE, 2 * D), jnp.float32),
    compiler_params=pltpu.CompilerParams(
        allow_input_fusion=[True, True, False, False]),
)


def kernel(sim_data, train_data, Em_table, Ed_table, W1, b1, W2, b2):
    edges = train_data.T
    w1bd = jnp.zeros((2 * D, 2 * D), jnp.bfloat16)
    w1bd = w1bd.at[:D, :HID].set(W1[:D].astype(jnp.bfloat16))
    w1bd = w1bd.at[D:, HID:].set(W1[D:].astype(jnp.bfloat16))
    b1p = jnp.concatenate([b1, jnp.zeros((HID,), jnp.float32)]).reshape(1, 2 * D)
    p = _proj(Em_table, Ed_table, w1bd, b1p)
    return _sc_score(sim_data, edges, p, W2.reshape(HID),
                     jnp.broadcast_to(b2, (L,)))
